# baseline (speedup 1.0000x reference)
import functools

import jax
import jax.numpy as jnp
from jax import lax
from jax.experimental import pallas as pl
from jax.experimental.pallas import tpu as pltpu
from jax.experimental.pallas import tpu_sc as plsc

HID = 768
INTER = 1024
NUM_ROUTED = 24
NUM_SHARED = 2
TOP_K = 6
RINTER = 256
N = 2048
NA = N * TOP_K
TILE = 128
NTILE = NA // TILE + NUM_ROUTED
NP = NTILE * TILE
NEG = -1e30


def _dispatch_kernel(
    xf_ref, gwt_ref, rb2_ref, sb2_ref, dest_ref, w6_ref, te_ref, bias_ref
):
    logits = jnp.dot(xf_ref[...], gwt_ref[...], preferred_element_type=jnp.float32)
    scores = jax.nn.sigmoid(logits)
    n, e = scores.shape
    col = lax.broadcasted_iota(jnp.int32, (n, e), 1)
    s = scores
    mask = jnp.zeros(scores.shape, dtype=jnp.bool_)
    picks = []
    for _ in range(TOP_K):
        m = jnp.max(s, axis=1, keepdims=True)
        is_max = s == m
        min_idx = jnp.min(jnp.where(is_max, col, e), axis=1, keepdims=True)
        pick = col == min_idx
        picks.append(pick)
        mask = mask | pick
        s = jnp.where(pick, NEG, s)
    sel = jnp.where(mask, scores, 0.0)
    denom = jnp.sum(sel, axis=1, keepdims=True) + 1e-9
    w_dense = sel / denom
    shared_b2 = jnp.sum(sb2_ref[...], axis=0, keepdims=True)
    bias_ref[...] = (
        jnp.dot(w_dense, rb2_ref[...], preferred_element_type=jnp.float32) + shared_b2
    )

    maskf = mask.astype(jnp.float32)
    ri = lax.broadcasted_iota(jnp.int32, (n, n), 0)
    ci = lax.broadcasted_iota(jnp.int32, (n, n), 1)
    tri = jnp.where(ci < ri, 1.0, 0.0)
    cum_excl = jnp.dot(tri, maskf, preferred_element_type=jnp.float32)
    counts = jnp.sum(maskf, axis=0, keepdims=True)
    tiles_e = jnp.floor((counts + (TILE - 1)) * (1.0 / TILE))
    ri24 = lax.broadcasted_iota(jnp.int32, (e, e), 0)
    ci24 = lax.broadcasted_iota(jnp.int32, (e, e), 1)
    ltri = jnp.where(ri24 < ci24, 1.0, 0.0)
    start_tiles = jnp.dot(tiles_e, ltri, preferred_element_type=jnp.float32)
    dest_mat = start_tiles * float(TILE) + cum_excl

    dcols, wcols = [], []
    for k in range(TOP_K):
        pk = picks[k]
        dcols.append(jnp.sum(jnp.where(pk, dest_mat, 0.0), axis=1, keepdims=True))
        wcols.append(jnp.sum(jnp.where(pk, w_dense, 0.0), axis=1, keepdims=True))
    dest_ref[...] = jnp.concatenate(dcols, axis=1).astype(jnp.int32)
    w6_ref[...] = jnp.concatenate(wcols, axis=1)

    ti = lax.broadcasted_iota(jnp.int32, (TILE, e), 0).astype(jnp.float32)
    ge = jnp.where(ti >= start_tiles, 1.0, 0.0)
    te_ref[...] = (jnp.sum(ge, axis=1, keepdims=True) - 1.0).astype(jnp.int32)


@functools.lru_cache(maxsize=None)
def _sc_mesh():
    return plsc.VectorSubcoreMesh(core_axis_name="c", subcore_axis_name="s")


def _sc_scatter(
    dest3, tok3, w63, zi, zf, stok_out, sw_out, idxv, tokv, wv, zvi, zvf, sh_tok, sh_w
):
    c = lax.axis_index("c")
    s = lax.axis_index("s")
    slot = NP // 16

    @pl.when(c == 0)
    def _():
        pltpu.sync_copy(zi.at[pl.ds(s * slot, slot)], zvi)
        pltpu.sync_copy(zf.at[pl.ds(s * slot, slot)], zvf)
        pltpu.sync_copy(zvi, sh_tok.at[pl.ds(s * slot, slot)])
        pltpu.sync_copy(zvf, sh_w.at[pl.ds(s * slot, slot)])
        plsc.subcore_barrier()
        pltpu.sync_copy(dest3.at[s], idxv)
        pltpu.sync_copy(tok3.at[s], tokv)
        pltpu.sync_copy(w63.at[s], wv)
        for j in range(NA // 16 // TILE):
            pltpu.sync_copy(tokv.at[j], sh_tok.at[idxv.at[j]], add=True)
            pltpu.sync_copy(wv.at[j], sh_w.at[idxv.at[j]], add=True)
        plsc.subcore_barrier()
        pltpu.sync_copy(sh_tok.at[pl.ds(s * slot, slot)], zvi)
        pltpu.sync_copy(sh_w.at[pl.ds(s * slot, slot)], zvf)
        pltpu.sync_copy(zvi, stok_out.at[pl.ds(s * slot, slot)])
        pltpu.sync_copy(zvf, sw_out.at[pl.ds(s * slot, slot)])


_GCH = 96


def _sc_gather(x_hbm, stok3, xs_out, idxv, rows_v):
    c = lax.axis_index("c")
    s = lax.axis_index("s")
    wid = s * 2 + c
    pltpu.sync_copy(stok3.at[wid], idxv)
    for j in range(NP // 32 // _GCH):
        pltpu.sync_copy(x_hbm.at[idxv.at[j]], rows_v)
        pltpu.sync_copy(rows_v, xs_out.at[pl.ds(wid * (NP // 32) + j * _GCH, _GCH)])


_HALF = HID // 2
_QTR = HID // 6


def _sc_combine(y_hbm, stok2, zrows, part_out, idxv, rows_v, wbuf, acc):
    c = lax.axis_index("c")
    s = lax.axis_index("s")
    pltpu.sync_copy(stok2.at[s], idxv)
    for p in range(3):
        pltpu.sync_copy(zrows, rows_v.at[pl.ds(0, 32)])
        for q in range(4):
            pltpu.sync_copy(
                rows_v.at[pl.ds(0, 32)], acc.at[pl.ds(s * (N // 16) + q * 32, 32)]
            )
        plsc.subcore_barrier()
        for j in range(NP // 16 // _GCH):
            pltpu.sync_copy(
                y_hbm.at[
                    pl.ds(s * (NP // 16) + j * _GCH, _GCH),
                    pl.ds(c * _HALF + p * _QTR, _QTR),
                ],
                rows_v,
            )
            pltpu.sync_copy(rows_v, acc.at[idxv.at[j]], add=True)
        plsc.subcore_barrier()
        pltpu.sync_copy(acc.at[pl.ds(s * (N // 16), N // 16)], wbuf)
        pltpu.sync_copy(
            wbuf, part_out.at[c, pl.ds(s * (N // 16), N // 16), pl.ds(p * _QTR, _QTR)]
        )


def _ffn_kernel(te_ref, xs_ref, w1_ref, b1_ref, w2_ref, swc_ref, y_ref):
    h = jnp.dot(xs_ref[...], w1_ref[0], preferred_element_type=jnp.float32)
    h = jax.nn.gelu(h + b1_ref[0])
    y_ref[...] = jnp.dot(h, w2_ref[0], preferred_element_type=jnp.float32) * swc_ref[...]


def _shared_kernel(x_ref, sw1_ref, sb1_ref, sw2_ref, part_ref, bias_ref, out_ref, sem):
    g = pl.program_id(0)

    @pl.when(g == 0)
    def _():
        pltpu.make_async_copy(bias_ref, out_ref, sem).start()
        pltpu.make_async_copy(bias_ref, out_ref, sem).wait()

    h = jnp.dot(x_ref[...], sw1_ref[0], preferred_element_type=jnp.float32)
    h = jax.nn.gelu(h + sb1_ref[0])
    out_ref[...] += jnp.dot(h, sw2_ref[0], preferred_element_type=jnp.float32)
    out_ref[:, pl.ds(g * _HALF, _HALF)] += part_ref[0]


def kernel(x, gate_W, sW1, sb1, sW2, sb2, rW1, rb1, rW2, rb2):
    b, s_, d = x.shape
    xf = x.reshape(-1, d)

    dest, w6, te2, bias_total = pl.pallas_call(
        _dispatch_kernel,
        out_shape=(
            jax.ShapeDtypeStruct((N, TOP_K), jnp.int32),
            jax.ShapeDtypeStruct((N, TOP_K), jnp.float32),
            jax.ShapeDtypeStruct((TILE, 1), jnp.int32),
            jax.ShapeDtypeStruct((N, HID), jnp.float32),
        ),
    )(xf, gate_W.T, rb2, sb2)

    dest3 = dest.reshape(16, NA // 16 // TILE, TILE)
    tok3 = jnp.repeat(jnp.arange(N, dtype=jnp.int32), TOP_K).reshape(
        16, NA // 16 // TILE, TILE
    )
    w63 = w6.reshape(16, NA // 16 // TILE, TILE)
    zi = jnp.zeros((NP,), jnp.int32)
    zf = jnp.zeros((NP,), jnp.float32)

    stok, sw = pl.kernel(
        _sc_scatter,
        out_type=(
            jax.ShapeDtypeStruct((NP,), jnp.int32),
            jax.ShapeDtypeStruct((NP,), jnp.float32),
        ),
        mesh=_sc_mesh(),
        scratch_types=[
            pltpu.VMEM((NA // 16 // TILE, TILE), jnp.int32),
            pltpu.VMEM((NA // 16 // TILE, TILE), jnp.int32),
            pltpu.VMEM((NA // 16 // TILE, TILE), jnp.float32),
            pltpu.VMEM((NP // 16,), jnp.int32),
            pltpu.VMEM((NP // 16,), jnp.float32),
            pltpu.VMEM_SHARED((NP,), jnp.int32),
            pltpu.VMEM_SHARED((NP,), jnp.float32),
        ],
    )(dest3, tok3, w63, zi, zf)

    stok3 = stok.reshape(32, NP // 32 // _GCH, _GCH)

    xs = pl.kernel(
        _sc_gather,
        out_type=jax.ShapeDtypeStruct((NP, HID), jnp.float32),
        mesh=_sc_mesh(),
        scratch_types=[
            pltpu.VMEM((NP // 32 // _GCH, _GCH), jnp.int32),
            pltpu.VMEM((_GCH, HID), jnp.float32),
        ],
    )(xf, stok3)

    te = te2.reshape(TILE)[:NTILE]
    swc = sw.reshape(NP, 1)
    rb1r = rb1.reshape(NUM_ROUTED, 1, RINTER)

    y = pl.pallas_call(
        _ffn_kernel,
        grid_spec=pltpu.PrefetchScalarGridSpec(
            num_scalar_prefetch=1,
            grid=(NTILE,),
            in_specs=[
                pl.BlockSpec((TILE, HID), lambda i, te_r: (i, 0)),
                pl.BlockSpec((1, HID, RINTER), lambda i, te_r: (te_r[i], 0, 0)),
                pl.BlockSpec((1, 1, RINTER), lambda i, te_r: (te_r[i], 0, 0)),
                pl.BlockSpec((1, RINTER, HID), lambda i, te_r: (te_r[i], 0, 0)),
                pl.BlockSpec((TILE, 1), lambda i, te_r: (i, 0)),
            ],
            out_specs=pl.BlockSpec((TILE, HID), lambda i, te_r: (i, 0)),
        ),
        out_shape=jax.ShapeDtypeStruct((NP, HID), jnp.float32),
    )(te, xs, rW1, rb1r, rW2, swc)

    zrows = jnp.zeros((32, _QTR), jnp.float32)
    stok2 = stok.reshape(16, NP // 16 // _GCH, _GCH)
    part = pl.kernel(
        _sc_combine,
        out_type=jax.ShapeDtypeStruct((2, N, _HALF), jnp.float32),
        mesh=_sc_mesh(),
        scratch_types=[
            pltpu.VMEM((NP // 16 // _GCH, _GCH), jnp.int32),
            pltpu.VMEM((_GCH, _QTR), jnp.float32),
            pltpu.VMEM((N // 16, _QTR), jnp.float32),
            pltpu.VMEM_SHARED((N, _QTR), jnp.float32),
        ],
    )(y, stok2, zrows)

    sb1r = sb1.reshape(NUM_SHARED, 1, INTER)
    out = pl.pallas_call(
        _shared_kernel,
        grid=(NUM_SHARED,),
        in_specs=[
            pl.BlockSpec((N, HID), lambda g: (0, 0)),
            pl.BlockSpec((1, HID, INTER), lambda g: (g, 0, 0)),
            pl.BlockSpec((1, 1, INTER), lambda g: (g, 0, 0)),
            pl.BlockSpec((1, INTER, HID), lambda g: (g, 0, 0)),
            pl.BlockSpec((1, N, _HALF), lambda g: (g, 0, 0)),
            pl.BlockSpec(memory_space=pl.ANY),
        ],
        out_specs=pl.BlockSpec((N, HID), lambda g: (0, 0)),
        out_shape=jax.ShapeDtypeStruct((N, HID), jnp.float32),
        scratch_shapes=[pltpu.SemaphoreType.DMA],
    )(xf, sW1, sb1r, sW2, part, bias_total)

    aux_loss = jnp.asarray(0.0, dtype=jnp.float32)
    return (out.reshape(b, s_, d), aux_loss)


# speedup vs baseline: 3.9398x; 3.9398x over previous
import jax
import jax.numpy as jnp
from jax.experimental import pallas as pl
from jax.experimental.pallas import tpu as pltpu

HID = 768
INTER = 1024
NUM_ROUTED = 24
NUM_SHARED = 2
TOP_K = 6
RINTER = 256
N_SHARED_CHUNK = NUM_SHARED * (INTER // RINTER)
N_CHUNK = NUM_ROUTED + N_SHARED_CHUNK
QUAD = 4
N_STEP = N_CHUNK // QUAD
N_ROUTED_STEP = NUM_ROUTED // QUAD


def _router_kernel(xf_ref, gwt_ref, rb2_ref, sb2_ref, w_ref, bias_ref):
    logits = jnp.dot(xf_ref[...], gwt_ref[...], preferred_element_type=jnp.float32)
    scores = jax.nn.sigmoid(logits)
    n, e = scores.shape
    col = jax.lax.broadcasted_iota(jnp.int32, (n, e), 1)
    s = scores
    mask = jnp.zeros(scores.shape, dtype=jnp.bool_)
    for _ in range(TOP_K):
        m = jnp.max(s, axis=1, keepdims=True)
        is_max = s == m
        min_idx = jnp.min(jnp.where(is_max, col, e), axis=1, keepdims=True)
        pick = col == min_idx
        mask = mask | pick
        s = jnp.where(pick, -jnp.inf, s)
    sel = jnp.where(mask, scores, 0.0)
    w = sel / (jnp.sum(sel, axis=1, keepdims=True) + 1e-9)
    w_ref[...] = w
    shared_b2 = jnp.sum(sb2_ref[...], axis=0, keepdims=True)
    bias_ref[...] = (
        jnp.dot(w, rb2_ref[...], preferred_element_type=jnp.float32) + shared_b2
    )


def _moe_kernel(
    w_ref, x_ref, rw1_ref, rw2_ref, sw1_ref, sw2_ref, cb1_ref, bias_ref, out_ref, sem
):
    g = pl.program_id(0)

    @pl.when(g == 0)
    def _():
        pltpu.make_async_copy(bias_ref, out_ref, sem).start()
        pltpu.make_async_copy(bias_ref, out_ref, sem).wait()

    routed = g < N_ROUTED_STEP
    xb = x_ref[...].astype(jnp.bfloat16)
    h_cols = []
    for i in range(QUAD):
        sl = slice(i * RINTER, (i + 1) * RINTER)
        w1_i = jnp.where(routed, rw1_ref[i], sw1_ref[0][:, sl]).astype(jnp.bfloat16)
        h_i = jnp.dot(xb, w1_i, preferred_element_type=jnp.float32)
        h_i = jax.nn.gelu(h_i + cb1_ref[0][:, sl]) * w_ref[i]
        h_cols.append(h_i.astype(jnp.bfloat16))
    h = jnp.concatenate(h_cols, axis=1)
    w2 = jnp.where(routed, rw2_ref[...].reshape(INTER, HID), sw2_ref[0]).astype(
        jnp.bfloat16
    )
    out_ref[...] += jnp.dot(h, w2, preferred_element_type=jnp.float32)


def kernel(x, gate_W, sW1, sb1, sW2, sb2, rW1, rb1, rW2, rb2):
    b, s, d = x.shape
    xf = x.reshape(-1, d)
    n = xf.shape[0]

    w_routed, bias_total = pl.pallas_call(
        _router_kernel,
        out_shape=(
            jax.ShapeDtypeStruct((n, NUM_ROUTED), jnp.float32),
            jax.ShapeDtypeStruct((n, HID), jnp.float32),
        ),
    )(xf, gate_W.T, rb2, sb2)

    sb1c = sb1.reshape(N_SHARED_CHUNK, RINTER)
    cb1 = jnp.concatenate([rb1, sb1c], axis=0).reshape(N_STEP, 1, INTER)
    w_full = jnp.concatenate(
        [w_routed, jnp.ones((n, N_SHARED_CHUNK), jnp.float32)], axis=1
    )
    w_full = w_full.T.reshape(N_CHUNK, n, 1)

    def routed_idx(g):
        return (jnp.minimum(g, N_ROUTED_STEP - 1), 0, 0)

    def shared_idx(g):
        return (jnp.maximum(g - N_ROUTED_STEP, 0), 0, 0)

    out = pl.pallas_call(
        _moe_kernel,
        grid=(N_STEP,),
        in_specs=[
            pl.BlockSpec((QUAD, n, 1), lambda g: (g, 0, 0)),
            pl.BlockSpec((n, HID), lambda g: (0, 0)),
            pl.BlockSpec((QUAD, HID, RINTER), routed_idx),
            pl.BlockSpec((QUAD, RINTER, HID), routed_idx),
            pl.BlockSpec((1, HID, INTER), shared_idx),
            pl.BlockSpec((1, INTER, HID), shared_idx),
            pl.BlockSpec((1, 1, INTER), lambda g: (g, 0, 0)),
            pl.BlockSpec(memory_space=pl.ANY),
        ],
        out_specs=pl.BlockSpec((n, HID), lambda g: (0, 0)),
        out_shape=jax.ShapeDtypeStruct((n, HID), jnp.float32),
        scratch_shapes=[pltpu.SemaphoreType.DMA],
    )(w_full, xf, rW1, rW2, sW1, sW2, cb1, bias_total)

    aux_loss = jnp.asarray(0.0, dtype=jnp.float32)
    return (out.reshape(b, s, d), aux_loss)


# speedup vs baseline: 5.1840x; 1.3158x over previous
import jax
import jax.numpy as jnp
from jax import lax
from jax.experimental import pallas as pl
from jax.experimental.pallas import tpu as pltpu

HID = 768
INTER = 1024
NUM_ROUTED = 24
NUM_SHARED = 2
TOP_K = 6
RINTER = 256
N_SHARED_CHUNK = NUM_SHARED * (INTER // RINTER)
N_CHUNK = NUM_ROUTED + N_SHARED_CHUNK
QUAD = 4
N_STEP = N_CHUNK // QUAD
N_ROUTED_STEP = NUM_ROUTED // QUAD


def _moe_kernel(
    x_ref,
    gw_ref,
    rb2_ref,
    sb2_ref,
    rw1_ref,
    rw2_ref,
    sw1_ref,
    sw2_ref,
    cb1_ref,
    out_ref,
    w_scr,
):
    g = pl.program_id(0)

    @pl.when(g == 0)
    def _():
        logits = lax.dot_general(
            x_ref[...],
            gw_ref[...],
            (((1,), (1,)), ((), ())),
            preferred_element_type=jnp.float32,
        )
        scores = jax.nn.sigmoid(logits)
        n, e = scores.shape
        col = lax.broadcasted_iota(jnp.int32, (n, e), 1)
        s = scores
        mask = jnp.zeros(scores.shape, dtype=jnp.bool_)
        for _ in range(TOP_K):
            m = jnp.max(s, axis=1, keepdims=True)
            is_max = s == m
            min_idx = jnp.min(jnp.where(is_max, col, e), axis=1, keepdims=True)
            pick = col == min_idx
            mask = mask | pick
            s = jnp.where(pick, -jnp.inf, s)
        sel = jnp.where(mask, scores, 0.0)
        w = sel / (jnp.sum(sel, axis=1, keepdims=True) + 1e-9)
        w_full = jnp.concatenate(
            [w, jnp.ones((n, N_SHARED_CHUNK), jnp.float32)], axis=1
        )
        for gg in range(N_STEP):
            w_scr[gg] = w_full[:, gg * QUAD : (gg + 1) * QUAD]
        shared_b2 = jnp.sum(sb2_ref[...], axis=0, keepdims=True)
        out_ref[...] = (
            jnp.dot(w, rb2_ref[...], preferred_element_type=jnp.float32) + shared_b2
        )

    routed = g < N_ROUTED_STEP
    wq = w_scr[g]
    h_cols = []
    for i in range(QUAD):
        sl = slice(i * RINTER, (i + 1) * RINTER)
        w1_i = jnp.where(routed, rw1_ref[i], sw1_ref[0][:, sl])
        h_i = jnp.dot(x_ref[...], w1_i, preferred_element_type=jnp.float32)
        h_i = jax.nn.gelu(h_i + cb1_ref[0][:, sl]) * wq[:, i : i + 1]
        h_cols.append(h_i)
    h = jnp.concatenate(h_cols, axis=1)
    w2 = jnp.where(routed, rw2_ref[...].reshape(INTER, HID), sw2_ref[0])
    out_ref[...] += jnp.dot(h, w2, preferred_element_type=jnp.float32)


def kernel(x, gate_W, sW1, sb1, sW2, sb2, rW1, rb1, rW2, rb2):
    b, s, d = x.shape
    xf = x.reshape(-1, d)
    n = xf.shape[0]

    sb1c = sb1.reshape(N_SHARED_CHUNK, RINTER)
    cb1 = jnp.concatenate([rb1, sb1c], axis=0).reshape(N_STEP, 1, INTER)

    def routed_idx(g):
        return (jnp.minimum(g, N_ROUTED_STEP - 1), 0, 0)

    def shared_idx(g):
        return (jnp.maximum(g - N_ROUTED_STEP, 0), 0, 0)

    out = pl.pallas_call(
        _moe_kernel,
        grid=(N_STEP,),
        in_specs=[
            pl.BlockSpec((n, HID), lambda g: (0, 0)),
            pl.BlockSpec((NUM_ROUTED, HID), lambda g: (0, 0)),
            pl.BlockSpec((NUM_ROUTED, HID), lambda g: (0, 0)),
            pl.BlockSpec((NUM_SHARED, HID), lambda g: (0, 0)),
            pl.BlockSpec((QUAD, HID, RINTER), routed_idx),
            pl.BlockSpec((QUAD, RINTER, HID), routed_idx),
            pl.BlockSpec((1, HID, INTER), shared_idx),
            pl.BlockSpec((1, INTER, HID), shared_idx),
            pl.BlockSpec((1, 1, INTER), lambda g: (g, 0, 0)),
        ],
        out_specs=pl.BlockSpec((n, HID), lambda g: (0, 0)),
        out_shape=jax.ShapeDtypeStruct((n, HID), jnp.float32),
        scratch_shapes=[pltpu.VMEM((N_STEP, n, QUAD), jnp.float32)],
        compiler_params=pltpu.CompilerParams(vmem_limit_bytes=100 * 1024 * 1024),
    )(xf, gate_W, rb2, sb2, rW1, rW2, sW1, sW2, cb1)

    aux_loss = jnp.asarray(0.0, dtype=jnp.float32)
    return (out.reshape(b, s, d), aux_loss)
